# baseline (speedup 1.0000x reference)
import functools

import jax
import jax.numpy as jnp
from jax import lax
from jax.experimental import pallas as pl
from jax.experimental.pallas import tpu as pltpu
from jax.experimental.pallas import tpu_sc as plsc

_CHUNK = 128


def kernel(labels, embedding_table, train):
    del train
    B = labels.shape[0]
    V, D = embedding_table.shape

    info = plsc.get_sparse_core_info()
    NC, NS = info.num_cores, info.num_subcores
    d_per_c = D // NC
    b_per_s = B // NS
    n_chunks = b_per_s // _CHUNK

    v_main = (V // 128) * 128
    BV = v_main + 128

    tail = embedding_table[v_main:].T
    tail = jnp.pad(tail, ((0, 0), (0, 128 - tail.shape[1])))

    mesh = plsc.VectorSubcoreMesh(core_axis_name="c", subcore_axis_name="s")

    @functools.partial(
        pl.kernel,
        mesh=mesh,
        out_type=jax.ShapeDtypeStruct((D, B), jnp.float32),
        scratch_types=[
            pltpu.VMEM((b_per_s,), jnp.int32),
            pltpu.VMEM((b_per_s,), jnp.float32),
            pltpu.VMEM((b_per_s,), jnp.float32),
            pltpu.VMEM_SHARED((BV,), jnp.float32),
            pltpu.VMEM_SHARED((BV,), jnp.float32),
            pltpu.SemaphoreType.DMA,
            pltpu.SemaphoreType.DMA,
            pltpu.SemaphoreType.DMA,
        ],
        compiler_params=pltpu.CompilerParams(needs_layout_passes=False),
    )
    def emb(idx_hbm, tab_hbm, tail_hbm, out_hbm, idx_v, bnc_a, bnc_b,
            buf_a, buf_b, row_sem, g_sem, w_sem):
        cid = lax.axis_index("c")
        sid = lax.axis_index("s")
        ibase = pl.multiple_of(sid * b_per_s, b_per_s)
        d0 = cid * d_per_c

        q = v_main // 4

        def stage(d, buf):
            for j in range(4):
                pltpu.async_copy(
                    tab_hbm.at[d].at[pl.ds(j * q, q)],
                    buf.at[pl.ds(j * q, q)],
                    row_sem,
                )
            pltpu.async_copy(
                tail_hbm.at[d], buf.at[pl.ds(v_main, 128)], row_sem
            )

        def wait_stage():
            pltpu.make_async_copy(
                tab_hbm.at[0].at[pl.ds(0, v_main)],
                buf_a.at[pl.ds(0, v_main)],
                row_sem,
            ).wait()
            pltpu.make_async_copy(
                tail_hbm.at[0], buf_a.at[pl.ds(v_main, 128)], row_sem
            ).wait()

        @pl.when(sid == 0)
        def _():
            stage(d0, buf_a)

        pltpu.sync_copy(idx_hbm.at[pl.ds(ibase, b_per_s)], idx_v)

        def do_feature(k, buf, other, bnc, kk):
            @pl.when(sid == 0)
            def _():
                wait_stage()

            plsc.subcore_barrier()

            @pl.when(jnp.logical_and(sid == 0, k + 1 < d_per_c))
            def _():
                stage(d0 + k + 1, other)

            @pl.when(kk > 0)
            def _():
                pltpu.make_async_copy(
                    idx_hbm.at[pl.ds(0, b_per_s)], bnc, w_sem
                ).wait()

            copies = []
            for m in range(n_chunks):
                copies.append(
                    pltpu.async_copy(
                        buf.at[idx_v.at[pl.ds(m * _CHUNK, _CHUNK)]],
                        bnc.at[pl.ds(m * _CHUNK, _CHUNK)],
                        g_sem,
                    )
                )
            for c in copies:
                c.wait()

            pltpu.async_copy(
                bnc, out_hbm.at[d0 + k, pl.ds(ibase, b_per_s)], w_sem
            )

        def pair(kk):
            do_feature(2 * kk, buf_a, buf_b, bnc_a, kk)
            do_feature(2 * kk + 1, buf_b, buf_a, bnc_b, kk)

        pl.loop(0, d_per_c // 2)(pair)

        pltpu.make_async_copy(idx_hbm.at[pl.ds(0, b_per_s)], bnc_a, w_sem).wait()
        pltpu.make_async_copy(idx_hbm.at[pl.ds(0, b_per_s)], bnc_b, w_sem).wait()

    return emb(labels, embedding_table.T, tail).T


# speedup vs baseline: 1.0053x; 1.0053x over previous
import functools

import jax
import jax.numpy as jnp
from jax import lax
from jax.experimental import pallas as pl
from jax.experimental.pallas import tpu as pltpu
from jax.experimental.pallas import tpu_sc as plsc

_CHUNK = 128


def kernel(labels, embedding_table, train):
    del train
    B = labels.shape[0]
    V, D = embedding_table.shape

    info = plsc.get_sparse_core_info()
    NC, NS = info.num_cores, info.num_subcores
    d_per_c = D // NC
    b_per_s = B // NS
    n_chunks = b_per_s // _CHUNK

    v_main = (V // 128) * 128
    BV = v_main + 128

    tail = embedding_table[v_main:].T
    tail = jnp.pad(tail, ((0, 0), (0, 128 - tail.shape[1])))

    mesh = plsc.VectorSubcoreMesh(core_axis_name="c", subcore_axis_name="s")

    @functools.partial(
        pl.kernel,
        mesh=mesh,
        out_type=jax.ShapeDtypeStruct((D, B), jnp.float32),
        scratch_types=[
            pltpu.VMEM((b_per_s,), jnp.int32),
            pltpu.VMEM((b_per_s,), jnp.float32),
            pltpu.VMEM((b_per_s,), jnp.float32),
            pltpu.VMEM_SHARED((BV,), jnp.float32),
            pltpu.VMEM_SHARED((BV,), jnp.float32),
            pltpu.SemaphoreType.DMA,
            pltpu.SemaphoreType.DMA,
            pltpu.SemaphoreType.DMA,
        ],
        compiler_params=pltpu.CompilerParams(needs_layout_passes=False),
    )
    def emb(idx_hbm, tab_hbm, tail_hbm, out_hbm, idx_v, bnc_a, bnc_b,
            buf_a, buf_b, row_sem, g_sem, w_sem):
        cid = lax.axis_index("c")
        sid = lax.axis_index("s")
        ibase = pl.multiple_of(sid * b_per_s, b_per_s)
        d0 = cid * d_per_c

        def stage(d, buf):
            pltpu.async_copy(
                tab_hbm.at[d].at[pl.ds(0, v_main)],
                buf.at[pl.ds(0, v_main)],
                row_sem,
            )
            pltpu.async_copy(
                tail_hbm.at[d], buf.at[pl.ds(v_main, 128)], row_sem
            )

        def wait_stage():
            pltpu.make_async_copy(
                tab_hbm.at[0].at[pl.ds(0, v_main)],
                buf_a.at[pl.ds(0, v_main)],
                row_sem,
            ).wait()
            pltpu.make_async_copy(
                tail_hbm.at[0], buf_a.at[pl.ds(v_main, 128)], row_sem
            ).wait()

        @pl.when(sid == 0)
        def _():
            stage(d0, buf_a)

        pltpu.sync_copy(idx_hbm.at[pl.ds(ibase, b_per_s)], idx_v)

        def do_feature(k, buf, other, bnc, kk):
            @pl.when(sid == 0)
            def _():
                wait_stage()

            plsc.subcore_barrier()

            @pl.when(jnp.logical_and(sid == 0, k + 1 < d_per_c))
            def _():
                stage(d0 + k + 1, other)

            @pl.when(kk > 0)
            def _():
                pltpu.make_async_copy(
                    idx_hbm.at[pl.ds(0, b_per_s)], bnc, w_sem
                ).wait()

            copies = []
            for m in range(n_chunks):
                copies.append(
                    pltpu.async_copy(
                        buf.at[idx_v.at[pl.ds(m * _CHUNK, _CHUNK)]],
                        bnc.at[pl.ds(m * _CHUNK, _CHUNK)],
                        g_sem,
                    )
                )
            for c in copies:
                c.wait()

            pltpu.async_copy(
                bnc, out_hbm.at[d0 + k, pl.ds(ibase, b_per_s)], w_sem
            )

        def pair(kk):
            do_feature(2 * kk, buf_a, buf_b, bnc_a, kk)
            do_feature(2 * kk + 1, buf_b, buf_a, bnc_b, kk)

        pl.loop(0, d_per_c // 2)(pair)

        pltpu.make_async_copy(idx_hbm.at[pl.ds(0, b_per_s)], bnc_a, w_sem).wait()
        pltpu.make_async_copy(idx_hbm.at[pl.ds(0, b_per_s)], bnc_b, w_sem).wait()

    return emb(labels, embedding_table.T, tail).T
